# baseline (speedup 1.0000x reference)
import functools

import jax
import jax.numpy as jnp
import numpy as np
from jax import lax
from jax.experimental import pallas as pl
from jax.experimental.pallas import tpu as pltpu
from jax.experimental.pallas import tpu_sc as plsc

N = 10000
E = 160000
H = 128
NH = 4
VOCAB = 1000
VOCABP = 1024
NP = 10240
NSUB = 16
ROWS_PER_SUB = NP // NSUB
CH = 128
ECHUNKS = 1280
EPAD = ECHUNKS * CH
NCHUNK_A = ECHUNKS // NSUB
NCHUNK_B = ECHUNKS // (2 * NSUB)
GRP = 8
NGRP_A = NCHUNK_A // GRP
NGRP_B = NCHUNK_B // GRP

F32 = jnp.float32


def _ln(x, g, b):
    mu = jnp.mean(x, axis=-1, keepdims=True)
    var = jnp.mean((x - mu) ** 2, axis=-1, keepdims=True)
    return (x - mu) / jnp.sqrt(var + 1e-5) * g + b


def _mha(q_in, k_in, v_in, wi_t, bi, wo_t, bo, sel, selt):
    q = [jnp.dot(t, wi_t[:, :H], preferred_element_type=F32) + bi[:, :H]
         for t in q_in]
    k = [jnp.dot(t, wi_t[:, H:2 * H], preferred_element_type=F32) + bi[:, H:2 * H]
         for t in k_in]
    v = [jnp.dot(t, wi_t[:, 2 * H:], preferred_element_type=F32) + bi[:, 2 * H:]
         for t in v_in]
    scale = 1.0 / np.sqrt(H / NH)
    s = [[jnp.dot(q[i] * k[j], sel, preferred_element_type=F32) * scale
          for j in range(3)] for i in range(3)]
    o = []
    for i in range(3):
        m = jnp.maximum(jnp.maximum(s[i][0], s[i][1]), s[i][2])
        e = [jnp.exp(s[i][j] - m) for j in range(3)]
        d = e[0] + e[1] + e[2]
        acc = None
        for j in range(3):
            w = jnp.dot(e[j] / d, selt, preferred_element_type=F32)
            acc = w * v[j] if acc is None else acc + w * v[j]
        o.append(acc)
    return [jnp.dot(t, wo_t, preferred_element_type=F32) + bo for t in o]


def _encode_body(enr_ref, sd_ref, st_ref, time_ref, numwt_ref, numb_ref,
                 lng_ref, lnb_ref, emb_ref, wih_ref, lb_ref, fawi_ref,
                 fabi_ref, fawo_ref, fabo_ref, convw_ref, convb_ref,
                 tawi_ref, tabi_ref, tawo_ref, tabo_ref, tew_ref, teb_ref,
                 sel_ref, selt_ref, x0_ref, x1_ref, x2_ref):
    R = enr_ref.shape[0]
    sel = sel_ref[...]
    selt = selt_ref[...]
    e_num = _ln(enr_ref[...] * numwt_ref[...] + numb_ref[...],
                lng_ref[...], lnb_ref[...])
    e_num = jnp.maximum(e_num, 0.0)
    g = sd_ref[...] * wih_ref[...] + lb_ref[...]
    gi, gg, go = g[:, :H], g[:, 2 * H:3 * H], g[:, 3 * H:]
    cst = jax.nn.sigmoid(gi) * jnp.tanh(gg)
    e_temp = jax.nn.sigmoid(go) * jnp.tanh(cst)
    iota = lax.broadcasted_iota(jnp.int32, (R, VOCABP), 1)
    oh = (iota == st_ref[...]).astype(F32)
    e_cat = jnp.dot(oh, emb_ref[...], preferred_element_type=F32)
    xs = [e_num, e_temp, e_cat]
    xs = _mha(xs, xs, xs, fawi_ref[...], fabi_ref[...], fawo_ref[...],
              fabo_ref[...], sel, selt)
    W = convw_ref[...]
    y = [[jnp.dot(xs[i], W[t], preferred_element_type=F32) for t in range(3)]
         for i in range(3)]
    cb = convb_ref[...]
    xc = [y[0][1] + y[1][2] + cb,
          y[0][0] + y[1][1] + y[2][2] + cb,
          y[1][0] + y[2][1] + cb]
    tm = time_ref[...]
    tew = tew_ref[...]
    teb = teb_ref[...]
    a = [xc[i] + tm[:, i:i + 1] * tew + teb for i in range(3)]
    out = _mha(a, a, xc, tawi_ref[...], tabi_ref[...], tawo_ref[...],
               tabo_ref[...], sel, selt)
    x0_ref[...] = out[0]
    x1_ref[...] = out[1]
    x2_ref[...] = out[2]


def _sage_body(s0_ref, s1_ref, s2a_ref, s2b_ref, ca_ref, cb_ref, x0_ref,
               x1_ref, x2_ref, wlt_ref, bl_ref, wrt_ref, g_ref, b_ref,
               h0_ref, h1_ref, h2_ref):
    cnt = jnp.maximum(ca_ref[...][:, :1] + cb_ref[...][:, :1], 1.0)
    ss = [s0_ref[...], s1_ref[...], s2a_ref[...] + s2b_ref[...]]
    xs = [x0_ref[...], x1_ref[...], x2_ref[...]]
    outs = [h0_ref, h1_ref, h2_ref]
    for i in range(3):
        y = (jnp.dot(ss[i] / cnt, wlt_ref[...], preferred_element_type=F32)
             + bl_ref[...]
             + jnp.dot(xs[i], wrt_ref[...], preferred_element_type=F32))
        outs[i][...] = jnp.maximum(_ln(y, g_ref[...], b_ref[...]), 0.0) + xs[i]


def _sage_out_body(s0_ref, s1_ref, s2a_ref, s2b_ref, ca_ref, cb_ref, x0_ref,
                   x1_ref, x2_ref, wlt_ref, bl_ref, wrt_ref, g_ref, b_ref,
                   owt_ref, ob_ref, out_ref):
    cnt = jnp.maximum(ca_ref[...][:, :1] + cb_ref[...][:, :1], 1.0)
    ss = [s0_ref[...], s1_ref[...], s2a_ref[...] + s2b_ref[...]]
    xs = [x0_ref[...], x1_ref[...], x2_ref[...]]
    cols = []
    for i in range(3):
        y = (jnp.dot(ss[i] / cnt, wlt_ref[...], preferred_element_type=F32)
             + bl_ref[...]
             + jnp.dot(xs[i], wrt_ref[...], preferred_element_type=F32))
        y = jnp.maximum(_ln(y, g_ref[...], b_ref[...]), 0.0) + xs[i]
        cols.append(jnp.dot(y, owt_ref[...], preferred_element_type=F32)
                    + ob_ref[...])
    out_ref[...] = jnp.concatenate(cols, axis=1)


def _make_seg_body(with_cnt):
    def _seg_body(x0, x1, x2, grouped, zrows, ones_hbm, *rest):
        if with_cnt:
            (s0, s1, s2a, s2b, cnta, cntb, iblk, isrc_b, idst_b, rows0,
             acc) = rest
        else:
            (s0, s1, s2a, s2b, iblk, isrc_b, idst_b, rows0, acc) = rest

        def stage_idx(i):
            for v in range(CH // 16):
                sl = pl.ds(v * 16, 16)
                isrc_b[sl] = iblk[i, sl]
                idst_b[sl] = iblk[GRP + i, sl]

        def stage_dst(i):
            for v in range(CH // 16):
                sl = pl.ds(v * 16, 16)
                idst_b[sl] = iblk[GRP + i, sl]
        c = lax.axis_index("c")
        sid = lax.axis_index("s")
        row0 = pl.multiple_of(sid * ROWS_PER_SUB, 8)

        def zero_acc():
            pltpu.sync_copy(zrows, acc.at[pl.ds(row0, ROWS_PER_SUB)])

        def writeback(dst_hbm):
            pltpu.sync_copy(acc.at[pl.ds(row0, ROWS_PER_SUB)],
                            dst_hbm.at[pl.ds(row0, ROWS_PER_SUB)])

        def run_phase(xtab, ngroups, grp0):
            def body(g, carry):
                pltpu.sync_copy(grouped.at[grp0 + g], iblk)
                for i in range(GRP):
                    stage_idx(i)
                    pltpu.sync_copy(xtab.at[isrc_b], rows0)
                    pltpu.sync_copy(rows0, acc.at[idst_b], add=True)
                return carry

            lax.fori_loop(0, ngroups, body, 0)

        zero_acc()
        plsc.subcore_barrier()

        grp_a = sid * NGRP_A

        @pl.when(c == 0)
        def _pa0():
            run_phase(x0, NGRP_A, grp_a)

        @pl.when(c == 1)
        def _pa1():
            run_phase(x1, NGRP_A, grp_a)

        plsc.subcore_barrier()

        @pl.when(c == 0)
        def _wb_a0():
            writeback(s0)

        @pl.when(c == 1)
        def _wb_a1():
            writeback(s1)

        zero_acc()
        plsc.subcore_barrier()

        grp_b = c * (NGRP_A * NSUB // 2) + sid * NGRP_B
        run_phase(x2, NGRP_B, grp_b)
        plsc.subcore_barrier()

        @pl.when(c == 0)
        def _wb_b0():
            writeback(s2a)

        @pl.when(c == 1)
        def _wb_b1():
            writeback(s2b)

        if not with_cnt:
            return

        zero_acc()
        ones_v = rows0
        pltpu.sync_copy(ones_hbm, ones_v)
        plsc.subcore_barrier()

        def body_c(g, carry):
            pltpu.sync_copy(grouped.at[grp_b + g], iblk)
            for i in range(GRP):
                stage_dst(i)
                pltpu.sync_copy(ones_v, acc.at[idst_b], add=True)
            return carry

        lax.fori_loop(0, NGRP_B, body_c, 0)
        plsc.subcore_barrier()

        @pl.when(c == 0)
        def _wb_c0():
            writeback(cnta)

        @pl.when(c == 1)
        def _wb_c1():
            writeback(cntb)

    return _seg_body


def _make_segsum(with_cnt):
    mesh = plsc.VectorSubcoreMesh(core_axis_name="c", subcore_axis_name="s")
    n_out = 6 if with_cnt else 4
    return pl.kernel(
        _make_seg_body(with_cnt),
        out_type=[jax.ShapeDtypeStruct((NP, H), F32)] * n_out,
        mesh=mesh,
        scratch_types=[
            pltpu.VMEM((2 * GRP, CH), jnp.int32),
            pltpu.VMEM((CH,), jnp.int32),
            pltpu.VMEM((CH,), jnp.int32),
            pltpu.VMEM((CH, H), F32),
            pltpu.VMEM_SHARED((NP, H), F32),
        ],
    )


def _full_spec(shape):
    nd = len(shape)
    return pl.BlockSpec(shape, lambda i, _nd=nd: (0,) * _nd)


def kernel(enrollment, start_date, study_type, time, edge_index, num_W, num_b,
           ln_num_g, ln_num_b, emb, lstm_Wih, lstm_Whh, lstm_bih, lstm_bhh,
           fa_Wi, fa_bi, fa_Wo, fa_bo, conv_W, conv_b, ta_Wi, ta_bi, ta_Wo,
           ta_bo, te_W, te_b, sage1_Wl, sage1_bl, sage1_Wr, sage2_Wl,
           sage2_bl, sage2_Wr, ln1_g, ln1_b, ln2_g, ln2_b, out_W, out_b):
    R = 400
    G = N // R

    src_g = jnp.concatenate(
        [edge_index[0].astype(jnp.int32),
         jnp.zeros((EPAD - E,), jnp.int32)]).reshape(-1, GRP, CH)
    dst_g = jnp.concatenate(
        [edge_index[1].astype(jnp.int32),
         N + jnp.arange(EPAD - E, dtype=jnp.int32) % (NP - N)]
    ).reshape(-1, GRP, CH)
    grouped = jnp.concatenate([src_g, dst_g], axis=1)
    enr = enrollment.reshape(N, 1)
    sd = start_date.reshape(N, 1)
    st = study_type.reshape(N, 1).astype(jnp.int32)

    numwt = num_W.reshape(1, H)
    numb = num_b.reshape(1, H)
    lng = ln_num_g.reshape(1, H)
    lnb = ln_num_b.reshape(1, H)
    emb_pad = jnp.pad(emb, ((0, VOCABP - VOCAB), (0, 0)))
    wih = lstm_Wih.reshape(1, 4 * H)
    lbias = (lstm_bih + lstm_bhh).reshape(1, 4 * H)
    fawi = fa_Wi.T
    fabi = fa_bi.reshape(1, 3 * H)
    fawo = fa_Wo.T
    fabo = fa_bo.reshape(1, H)
    convw = jnp.transpose(conv_W, (2, 1, 0))
    convb = conv_b.reshape(1, H)
    tawi = ta_Wi.T
    tabi = ta_bi.reshape(1, 3 * H)
    tawo = ta_Wo.T
    tabo = ta_bo.reshape(1, H)
    tew = te_W.reshape(1, H)
    teb = te_b.reshape(1, H)
    sel = (jnp.arange(H)[:, None] // (H // NH)
           == jnp.arange(NH)[None, :]).astype(F32)
    selt = sel.T

    row_spec = lambda w: pl.BlockSpec((R, w), lambda i: (i, 0))
    weight_args = [numwt, numb, lng, lnb, emb_pad, wih, lbias, fawi, fabi,
                   fawo, fabo, convw, convb, tawi, tabi, tawo, tabo, tew,
                   teb, sel, selt]
    x0, x1, x2 = pl.pallas_call(
        _encode_body,
        grid=(G,),
        in_specs=[row_spec(1), row_spec(1), row_spec(1), row_spec(3)]
        + [_full_spec(w.shape) for w in weight_args],
        out_specs=[row_spec(H)] * 3,
        out_shape=[jax.ShapeDtypeStruct((NP, H), F32)] * 3,
    )(enr, sd, st, time, *weight_args)

    zrows = jnp.zeros((ROWS_PER_SUB, H), F32)
    ones_rows = jnp.ones((CH, H), F32)

    s0, s1, s2a, s2b, cnta, cntb = _make_segsum(True)(
        x0, x1, x2, grouped, zrows, ones_rows)

    sage1_w = [sage1_Wl.T, sage1_bl.reshape(1, H), sage1_Wr.T,
               ln1_g.reshape(1, H), ln1_b.reshape(1, H)]
    h0, h1, h2 = pl.pallas_call(
        _sage_body,
        grid=(G,),
        in_specs=[row_spec(H)] * 9
        + [_full_spec(w.shape) for w in sage1_w],
        out_specs=[row_spec(H)] * 3,
        out_shape=[jax.ShapeDtypeStruct((NP, H), F32)] * 3,
    )(s0, s1, s2a, s2b, cnta, cntb, x0, x1, x2, *sage1_w)

    t0, t1, t2a, t2b = _make_segsum(False)(
        h0, h1, h2, grouped, zrows, ones_rows)

    sage2_w = [sage2_Wl.T, sage2_bl.reshape(1, H), sage2_Wr.T,
               ln2_g.reshape(1, H), ln2_b.reshape(1, H),
               out_W.T, out_b.reshape(1, H)]
    out = pl.pallas_call(
        _sage_out_body,
        grid=(G,),
        in_specs=[row_spec(H)] * 9
        + [_full_spec(w.shape) for w in sage2_w],
        out_specs=row_spec(3 * H),
        out_shape=jax.ShapeDtypeStruct((N, 3 * H), F32),
    )(t0, t1, t2a, t2b, cnta, cntb, h0, h1, h2, *sage2_w)

    return out.reshape(N, 3, H)


# speedup vs baseline: 1.4397x; 1.4397x over previous
import functools

import jax
import jax.numpy as jnp
import numpy as np
from jax import lax
from jax.experimental import pallas as pl
from jax.experimental.pallas import tpu as pltpu
from jax.experimental.pallas import tpu_sc as plsc

N = 10000
E = 160000
H = 128
NH = 4
VOCAB = 1000
VOCABP = 1024
NP = 10240
NSUB = 16
ROWS_PER_SUB = NP // NSUB
CH = 128
ECHUNKS = 1280
EPAD = ECHUNKS * CH
NCHUNK_A = ECHUNKS // NSUB
NCHUNK_B = ECHUNKS // (2 * NSUB)
GRP = 8
NGRP_A = NCHUNK_A // GRP
NGRP_B = NCHUNK_B // GRP

F32 = jnp.float32


def _ln(x, g, b):
    mu = jnp.mean(x, axis=-1, keepdims=True)
    var = jnp.mean((x - mu) ** 2, axis=-1, keepdims=True)
    return (x - mu) / jnp.sqrt(var + 1e-5) * g + b


def _mha(q_in, k_in, v_in, wi_t, bi, wo_t, bo, sel, selt):
    q = [jnp.dot(t, wi_t[:, :H], preferred_element_type=F32) + bi[:, :H]
         for t in q_in]
    k = [jnp.dot(t, wi_t[:, H:2 * H], preferred_element_type=F32) + bi[:, H:2 * H]
         for t in k_in]
    v = [jnp.dot(t, wi_t[:, 2 * H:], preferred_element_type=F32) + bi[:, 2 * H:]
         for t in v_in]
    scale = 1.0 / np.sqrt(H / NH)
    s = [[jnp.dot(q[i] * k[j], sel, preferred_element_type=F32) * scale
          for j in range(3)] for i in range(3)]
    o = []
    for i in range(3):
        m = jnp.maximum(jnp.maximum(s[i][0], s[i][1]), s[i][2])
        e = [jnp.exp(s[i][j] - m) for j in range(3)]
        d = e[0] + e[1] + e[2]
        acc = None
        for j in range(3):
            w = jnp.dot(e[j] / d, selt, preferred_element_type=F32)
            acc = w * v[j] if acc is None else acc + w * v[j]
        o.append(acc)
    return [jnp.dot(t, wo_t, preferred_element_type=F32) + bo for t in o]


def _encode_body(enr_ref, sd_ref, st_ref, time_ref, numwt_ref, numb_ref,
                 lng_ref, lnb_ref, emb_ref, wih_ref, lb_ref, fawi_ref,
                 fabi_ref, fawo_ref, fabo_ref, convw_ref, convb_ref,
                 tawi_ref, tabi_ref, tawo_ref, tabo_ref, tew_ref, teb_ref,
                 sel_ref, selt_ref, x0_ref, x1_ref, x2_ref):
    R = enr_ref.shape[0]
    sel = sel_ref[...]
    selt = selt_ref[...]
    e_num = _ln(enr_ref[...] * numwt_ref[...] + numb_ref[...],
                lng_ref[...], lnb_ref[...])
    e_num = jnp.maximum(e_num, 0.0)
    g = sd_ref[...] * wih_ref[...] + lb_ref[...]
    gi, gg, go = g[:, :H], g[:, 2 * H:3 * H], g[:, 3 * H:]
    cst = jax.nn.sigmoid(gi) * jnp.tanh(gg)
    e_temp = jax.nn.sigmoid(go) * jnp.tanh(cst)
    iota = lax.broadcasted_iota(jnp.int32, (R, VOCABP), 1)
    oh = (iota == st_ref[...]).astype(F32)
    e_cat = jnp.dot(oh, emb_ref[...], preferred_element_type=F32)
    xs = [e_num, e_temp, e_cat]
    xs = _mha(xs, xs, xs, fawi_ref[...], fabi_ref[...], fawo_ref[...],
              fabo_ref[...], sel, selt)
    W = convw_ref[...]
    y = [[jnp.dot(xs[i], W[t], preferred_element_type=F32) for t in range(3)]
         for i in range(3)]
    cb = convb_ref[...]
    xc = [y[0][1] + y[1][2] + cb,
          y[0][0] + y[1][1] + y[2][2] + cb,
          y[1][0] + y[2][1] + cb]
    tm = time_ref[...]
    tew = tew_ref[...]
    teb = teb_ref[...]
    a = [xc[i] + tm[:, i:i + 1] * tew + teb for i in range(3)]
    out = _mha(a, a, xc, tawi_ref[...], tabi_ref[...], tawo_ref[...],
               tabo_ref[...], sel, selt)
    x0_ref[...] = out[0]
    x1_ref[...] = out[1]
    x2_ref[...] = out[2]


def _sage_body(s0_ref, s1_ref, s2a_ref, s2b_ref, ca_ref, cb_ref, x0_ref,
               x1_ref, x2_ref, wlt_ref, bl_ref, wrt_ref, g_ref, b_ref,
               h0_ref, h1_ref, h2_ref):
    cnt = jnp.maximum(ca_ref[...][:, :1] + cb_ref[...][:, :1], 1.0)
    ss = [s0_ref[...], s1_ref[...], s2a_ref[...] + s2b_ref[...]]
    xs = [x0_ref[...], x1_ref[...], x2_ref[...]]
    outs = [h0_ref, h1_ref, h2_ref]
    for i in range(3):
        y = (jnp.dot(ss[i] / cnt, wlt_ref[...], preferred_element_type=F32)
             + bl_ref[...]
             + jnp.dot(xs[i], wrt_ref[...], preferred_element_type=F32))
        outs[i][...] = jnp.maximum(_ln(y, g_ref[...], b_ref[...]), 0.0) + xs[i]


def _sage_out_body(s0_ref, s1_ref, s2a_ref, s2b_ref, ca_ref, cb_ref, x0_ref,
                   x1_ref, x2_ref, wlt_ref, bl_ref, wrt_ref, g_ref, b_ref,
                   owt_ref, ob_ref, out_ref):
    cnt = jnp.maximum(ca_ref[...][:, :1] + cb_ref[...][:, :1], 1.0)
    ss = [s0_ref[...], s1_ref[...], s2a_ref[...] + s2b_ref[...]]
    xs = [x0_ref[...], x1_ref[...], x2_ref[...]]
    cols = []
    for i in range(3):
        y = (jnp.dot(ss[i] / cnt, wlt_ref[...], preferred_element_type=F32)
             + bl_ref[...]
             + jnp.dot(xs[i], wrt_ref[...], preferred_element_type=F32))
        y = jnp.maximum(_ln(y, g_ref[...], b_ref[...]), 0.0) + xs[i]
        cols.append(jnp.dot(y, owt_ref[...], preferred_element_type=F32)
                    + ob_ref[...])
    out_ref[...] = jnp.concatenate(cols, axis=1)


def _make_seg_body(with_cnt):
    def _seg_body(x0, x1, x2, src1d, dst1d, zrows, ones_hbm, *rest):
        if with_cnt:
            (s0, s1, s2a, s2b, cnta, cntb, iblk, isrc_b, idst_b, rows0,
             acc) = rest
        else:
            (s0, s1, s2a, s2b, iblk, isrc_b, idst_b, rows0, acc) = rest

        def stage_idx(i):
            for v in range(CH // 16):
                sl = pl.ds(v * 16, 16)
                isrc_b[sl] = iblk[i, sl]
                idst_b[sl] = iblk[GRP + i, sl]

        def stage_dst(i):
            for v in range(CH // 16):
                sl = pl.ds(v * 16, 16)
                idst_b[sl] = iblk[GRP + i, sl]
        c = lax.axis_index("c")
        sid = lax.axis_index("s")
        row0 = pl.multiple_of(sid * ROWS_PER_SUB, 8)

        def zero_acc():
            pltpu.sync_copy(zrows, acc.at[pl.ds(row0, ROWS_PER_SUB)])

        def writeback(dst_hbm):
            pltpu.sync_copy(acc.at[pl.ds(row0, ROWS_PER_SUB)],
                            dst_hbm.at[pl.ds(row0, ROWS_PER_SUB)])

        def run_phase(xtab, nchunks, blk0):
            def body(j, carry):
                off = pl.multiple_of(blk0 + j * CH, 8)
                pltpu.sync_copy(src1d.at[pl.ds(off, CH)], isrc_b)
                pltpu.sync_copy(dst1d.at[pl.ds(off, CH)], idst_b)
                pltpu.sync_copy(xtab.at[isrc_b], rows0)
                pltpu.sync_copy(rows0, acc.at[idst_b], add=True)
                return carry

            lax.fori_loop(0, nchunks, body, 0)

        zero_acc()
        plsc.subcore_barrier()

        blk_a = sid * NCHUNK_A * CH

        @pl.when(c == 0)
        def _pa0():
            run_phase(x0, NCHUNK_A, blk_a)

        @pl.when(c == 1)
        def _pa1():
            run_phase(x1, NCHUNK_A, blk_a)

        plsc.subcore_barrier()

        @pl.when(c == 0)
        def _wb_a0():
            writeback(s0)

        @pl.when(c == 1)
        def _wb_a1():
            writeback(s1)

        zero_acc()
        plsc.subcore_barrier()

        blk_b = (c * (ECHUNKS // 2) + sid * NCHUNK_B) * CH
        run_phase(x2, NCHUNK_B, blk_b)
        plsc.subcore_barrier()

        @pl.when(c == 0)
        def _wb_b0():
            writeback(s2a)

        @pl.when(c == 1)
        def _wb_b1():
            writeback(s2b)

        if not with_cnt:
            return

        zero_acc()
        ones_v = rows0
        pltpu.sync_copy(ones_hbm, ones_v)
        plsc.subcore_barrier()

        def body_c(j, carry):
            off = pl.multiple_of(blk_b + j * CH, 8)
            pltpu.sync_copy(dst1d.at[pl.ds(off, CH)], idst_b)
            pltpu.sync_copy(ones_v, acc.at[idst_b], add=True)
            return carry

        lax.fori_loop(0, NCHUNK_B, body_c, 0)
        plsc.subcore_barrier()

        @pl.when(c == 0)
        def _wb_c0():
            writeback(cnta)

        @pl.when(c == 1)
        def _wb_c1():
            writeback(cntb)

    return _seg_body


def _make_segsum(with_cnt):
    mesh = plsc.VectorSubcoreMesh(core_axis_name="c", subcore_axis_name="s")
    n_out = 6 if with_cnt else 4
    return pl.kernel(
        _make_seg_body(with_cnt),
        out_type=[jax.ShapeDtypeStruct((NP, H), F32)] * n_out,
        mesh=mesh,
        scratch_types=[
            pltpu.VMEM((2 * GRP, CH), jnp.int32),
            pltpu.VMEM((CH,), jnp.int32),
            pltpu.VMEM((CH,), jnp.int32),
            pltpu.VMEM((CH, H), F32),
            pltpu.VMEM_SHARED((NP, H), F32),
        ],
    )


def _full_spec(shape):
    nd = len(shape)
    return pl.BlockSpec(shape, lambda i, _nd=nd: (0,) * _nd)


def kernel(enrollment, start_date, study_type, time, edge_index, num_W, num_b,
           ln_num_g, ln_num_b, emb, lstm_Wih, lstm_Whh, lstm_bih, lstm_bhh,
           fa_Wi, fa_bi, fa_Wo, fa_bo, conv_W, conv_b, ta_Wi, ta_bi, ta_Wo,
           ta_bo, te_W, te_b, sage1_Wl, sage1_bl, sage1_Wr, sage2_Wl,
           sage2_bl, sage2_Wr, ln1_g, ln1_b, ln2_g, ln2_b, out_W, out_b):
    R = 400
    G = N // R

    src1d = jnp.concatenate(
        [edge_index[0].astype(jnp.int32),
         jnp.arange(EPAD - E, dtype=jnp.int32) % N])
    dst1d = jnp.concatenate(
        [edge_index[1].astype(jnp.int32),
         N + jnp.arange(EPAD - E, dtype=jnp.int32) % (NP - N)])
    enr = enrollment.reshape(N, 1)
    sd = start_date.reshape(N, 1)
    st = study_type.reshape(N, 1).astype(jnp.int32)

    numwt = num_W.reshape(1, H)
    numb = num_b.reshape(1, H)
    lng = ln_num_g.reshape(1, H)
    lnb = ln_num_b.reshape(1, H)
    emb_pad = jnp.pad(emb, ((0, VOCABP - VOCAB), (0, 0)))
    wih = lstm_Wih.reshape(1, 4 * H)
    lbias = (lstm_bih + lstm_bhh).reshape(1, 4 * H)
    fawi = fa_Wi.T
    fabi = fa_bi.reshape(1, 3 * H)
    fawo = fa_Wo.T
    fabo = fa_bo.reshape(1, H)
    convw = jnp.transpose(conv_W, (2, 1, 0))
    convb = conv_b.reshape(1, H)
    tawi = ta_Wi.T
    tabi = ta_bi.reshape(1, 3 * H)
    tawo = ta_Wo.T
    tabo = ta_bo.reshape(1, H)
    tew = te_W.reshape(1, H)
    teb = te_b.reshape(1, H)
    sel = (jnp.arange(H)[:, None] // (H // NH)
           == jnp.arange(NH)[None, :]).astype(F32)
    selt = sel.T

    row_spec = lambda w: pl.BlockSpec((R, w), lambda i: (i, 0))
    weight_args = [numwt, numb, lng, lnb, emb_pad, wih, lbias, fawi, fabi,
                   fawo, fabo, convw, convb, tawi, tabi, tawo, tabo, tew,
                   teb, sel, selt]
    x0, x1, x2 = pl.pallas_call(
        _encode_body,
        grid=(G,),
        in_specs=[row_spec(1), row_spec(1), row_spec(1), row_spec(3)]
        + [_full_spec(w.shape) for w in weight_args],
        out_specs=[row_spec(H)] * 3,
        out_shape=[jax.ShapeDtypeStruct((NP, H), F32)] * 3,
    )(enr, sd, st, time, *weight_args)

    zrows = jnp.zeros((ROWS_PER_SUB, H), F32)
    ones_rows = jnp.ones((CH, H), F32)

    s0, s1, s2a, s2b, cnta, cntb = _make_segsum(True)(
        x0, x1, x2, src1d, dst1d, zrows, ones_rows)

    sage1_w = [sage1_Wl.T, sage1_bl.reshape(1, H), sage1_Wr.T,
               ln1_g.reshape(1, H), ln1_b.reshape(1, H)]
    h0, h1, h2 = pl.pallas_call(
        _sage_body,
        grid=(G,),
        in_specs=[row_spec(H)] * 9
        + [_full_spec(w.shape) for w in sage1_w],
        out_specs=[row_spec(H)] * 3,
        out_shape=[jax.ShapeDtypeStruct((NP, H), F32)] * 3,
    )(s0, s1, s2a, s2b, cnta, cntb, x0, x1, x2, *sage1_w)

    t0, t1, t2a, t2b = _make_segsum(False)(
        h0, h1, h2, src1d, dst1d, zrows, ones_rows)

    sage2_w = [sage2_Wl.T, sage2_bl.reshape(1, H), sage2_Wr.T,
               ln2_g.reshape(1, H), ln2_b.reshape(1, H),
               out_W.T, out_b.reshape(1, H)]
    out = pl.pallas_call(
        _sage_out_body,
        grid=(G,),
        in_specs=[row_spec(H)] * 9
        + [_full_spec(w.shape) for w in sage2_w],
        out_specs=row_spec(3 * H),
        out_shape=jax.ShapeDtypeStruct((N, 3 * H), F32),
    )(t0, t1, t2a, t2b, cnta, cntb, h0, h1, h2, *sage2_w)

    return out.reshape(N, 3, H)


# speedup vs baseline: 1.7520x; 1.2169x over previous
import functools

import jax
import jax.numpy as jnp
import numpy as np
from jax import lax
from jax.experimental import pallas as pl
from jax.experimental.pallas import tpu as pltpu
from jax.experimental.pallas import tpu_sc as plsc

N = 10000
E = 160000
H = 128
NH = 4
VOCAB = 1000
VOCABP = 1024
NP = 10240
NSUB = 16
ROWS_PER_SUB = NP // NSUB
CH = 256
ECHUNKS = 640
EPAD = ECHUNKS * CH
NCHUNK_A = ECHUNKS // NSUB
NCHUNK_B = ECHUNKS // (2 * NSUB)
GRP = 8
NGRP_A = NCHUNK_A // GRP
NGRP_B = NCHUNK_B // GRP

F32 = jnp.float32


def _ln(x, g, b):
    mu = jnp.mean(x, axis=-1, keepdims=True)
    var = jnp.mean((x - mu) ** 2, axis=-1, keepdims=True)
    return (x - mu) / jnp.sqrt(var + 1e-5) * g + b


def _mha(q_in, k_in, v_in, wi_t, bi, wo_t, bo, sel, selt):
    q = [jnp.dot(t, wi_t[:, :H], preferred_element_type=F32) + bi[:, :H]
         for t in q_in]
    k = [jnp.dot(t, wi_t[:, H:2 * H], preferred_element_type=F32) + bi[:, H:2 * H]
         for t in k_in]
    v = [jnp.dot(t, wi_t[:, 2 * H:], preferred_element_type=F32) + bi[:, 2 * H:]
         for t in v_in]
    scale = 1.0 / np.sqrt(H / NH)
    s = [[jnp.dot(q[i] * k[j], sel, preferred_element_type=F32) * scale
          for j in range(3)] for i in range(3)]
    o = []
    for i in range(3):
        m = jnp.maximum(jnp.maximum(s[i][0], s[i][1]), s[i][2])
        e = [jnp.exp(s[i][j] - m) for j in range(3)]
        d = e[0] + e[1] + e[2]
        acc = None
        for j in range(3):
            w = jnp.dot(e[j] / d, selt, preferred_element_type=F32)
            acc = w * v[j] if acc is None else acc + w * v[j]
        o.append(acc)
    return [jnp.dot(t, wo_t, preferred_element_type=F32) + bo for t in o]


def _encode_body(enr_ref, sd_ref, st_ref, time_ref, numwt_ref, numb_ref,
                 lng_ref, lnb_ref, emb_ref, wih_ref, lb_ref, fawi_ref,
                 fabi_ref, fawo_ref, fabo_ref, convw_ref, convb_ref,
                 tawi_ref, tabi_ref, tawo_ref, tabo_ref, tew_ref, teb_ref,
                 sel_ref, selt_ref, x0_ref, x1_ref, x2_ref):
    R = enr_ref.shape[0]
    sel = sel_ref[...]
    selt = selt_ref[...]
    e_num = _ln(enr_ref[...] * numwt_ref[...] + numb_ref[...],
                lng_ref[...], lnb_ref[...])
    e_num = jnp.maximum(e_num, 0.0)
    g = sd_ref[...] * wih_ref[...] + lb_ref[...]
    gi, gg, go = g[:, :H], g[:, 2 * H:3 * H], g[:, 3 * H:]
    cst = jax.nn.sigmoid(gi) * jnp.tanh(gg)
    e_temp = jax.nn.sigmoid(go) * jnp.tanh(cst)
    iota = lax.broadcasted_iota(jnp.int32, (R, VOCABP), 1)
    oh = (iota == st_ref[...]).astype(F32)
    e_cat = jnp.dot(oh, emb_ref[...], preferred_element_type=F32)
    xs = [e_num, e_temp, e_cat]
    xs = _mha(xs, xs, xs, fawi_ref[...], fabi_ref[...], fawo_ref[...],
              fabo_ref[...], sel, selt)
    W = convw_ref[...]
    y = [[jnp.dot(xs[i], W[t], preferred_element_type=F32) for t in range(3)]
         for i in range(3)]
    cb = convb_ref[...]
    xc = [y[0][1] + y[1][2] + cb,
          y[0][0] + y[1][1] + y[2][2] + cb,
          y[1][0] + y[2][1] + cb]
    tm = time_ref[...]
    tew = tew_ref[...]
    teb = teb_ref[...]
    a = [xc[i] + tm[:, i:i + 1] * tew + teb for i in range(3)]
    out = _mha(a, a, xc, tawi_ref[...], tabi_ref[...], tawo_ref[...],
               tabo_ref[...], sel, selt)
    x0_ref[...] = out[0]
    x1_ref[...] = out[1]
    x2_ref[...] = out[2]


def _sage_body(s0_ref, s1_ref, s2a_ref, s2b_ref, ca_ref, cb_ref, x0_ref,
               x1_ref, x2_ref, wlt_ref, bl_ref, wrt_ref, g_ref, b_ref,
               h0_ref, h1_ref, h2_ref):
    cnt = jnp.maximum(ca_ref[...][:, :1] + cb_ref[...][:, :1], 1.0)
    ss = [s0_ref[...], s1_ref[...], s2a_ref[...] + s2b_ref[...]]
    xs = [x0_ref[...], x1_ref[...], x2_ref[...]]
    outs = [h0_ref, h1_ref, h2_ref]
    for i in range(3):
        y = (jnp.dot(ss[i] / cnt, wlt_ref[...], preferred_element_type=F32)
             + bl_ref[...]
             + jnp.dot(xs[i], wrt_ref[...], preferred_element_type=F32))
        outs[i][...] = jnp.maximum(_ln(y, g_ref[...], b_ref[...]), 0.0) + xs[i]


def _sage_out_body(s0_ref, s1_ref, s2a_ref, s2b_ref, ca_ref, cb_ref, x0_ref,
                   x1_ref, x2_ref, wlt_ref, bl_ref, wrt_ref, g_ref, b_ref,
                   owt_ref, ob_ref, out_ref):
    cnt = jnp.maximum(ca_ref[...][:, :1] + cb_ref[...][:, :1], 1.0)
    ss = [s0_ref[...], s1_ref[...], s2a_ref[...] + s2b_ref[...]]
    xs = [x0_ref[...], x1_ref[...], x2_ref[...]]
    cols = []
    for i in range(3):
        y = (jnp.dot(ss[i] / cnt, wlt_ref[...], preferred_element_type=F32)
             + bl_ref[...]
             + jnp.dot(xs[i], wrt_ref[...], preferred_element_type=F32))
        y = jnp.maximum(_ln(y, g_ref[...], b_ref[...]), 0.0) + xs[i]
        cols.append(jnp.dot(y, owt_ref[...], preferred_element_type=F32)
                    + ob_ref[...])
    out_ref[...] = jnp.concatenate(cols, axis=1)


def _make_seg_body(with_cnt):
    def _seg_body(x0, x1, x2, src1d, dst1d, zrows, ones_hbm, *rest):
        if with_cnt:
            (s0, s1, s2a, s2b, cnta, cntb, iblk, isrc_b, idst_b, rows0,
             acc) = rest
        else:
            (s0, s1, s2a, s2b, iblk, isrc_b, idst_b, rows0, acc) = rest

        def stage_idx(i):
            for v in range(CH // 16):
                sl = pl.ds(v * 16, 16)
                isrc_b[sl] = iblk[i, sl]
                idst_b[sl] = iblk[GRP + i, sl]

        def stage_dst(i):
            for v in range(CH // 16):
                sl = pl.ds(v * 16, 16)
                idst_b[sl] = iblk[GRP + i, sl]
        c = lax.axis_index("c")
        sid = lax.axis_index("s")
        row0 = pl.multiple_of(sid * ROWS_PER_SUB, 8)

        def zero_acc():
            pltpu.sync_copy(zrows, acc.at[pl.ds(row0, ROWS_PER_SUB)])

        def writeback(dst_hbm):
            pltpu.sync_copy(acc.at[pl.ds(row0, ROWS_PER_SUB)],
                            dst_hbm.at[pl.ds(row0, ROWS_PER_SUB)])

        def run_phase(xtab, nchunks, blk0):
            def body(j, carry):
                off = pl.multiple_of(blk0 + j * CH, 8)
                pltpu.sync_copy(src1d.at[pl.ds(off, CH)], isrc_b)
                pltpu.sync_copy(dst1d.at[pl.ds(off, CH)], idst_b)
                pltpu.sync_copy(xtab.at[isrc_b], rows0)
                pltpu.sync_copy(rows0, acc.at[idst_b], add=True)
                return carry

            lax.fori_loop(0, nchunks, body, 0)

        zero_acc()
        plsc.subcore_barrier()

        blk_a = sid * NCHUNK_A * CH

        @pl.when(c == 0)
        def _pa0():
            run_phase(x0, NCHUNK_A, blk_a)

        @pl.when(c == 1)
        def _pa1():
            run_phase(x1, NCHUNK_A, blk_a)

        plsc.subcore_barrier()

        @pl.when(c == 0)
        def _wb_a0():
            writeback(s0)

        @pl.when(c == 1)
        def _wb_a1():
            writeback(s1)

        zero_acc()
        plsc.subcore_barrier()

        blk_b = (c * (ECHUNKS // 2) + sid * NCHUNK_B) * CH
        run_phase(x2, NCHUNK_B, blk_b)
        plsc.subcore_barrier()

        @pl.when(c == 0)
        def _wb_b0():
            writeback(s2a)

        @pl.when(c == 1)
        def _wb_b1():
            writeback(s2b)

        if not with_cnt:
            return

        zero_acc()
        ones_v = rows0
        pltpu.sync_copy(ones_hbm, ones_v)
        plsc.subcore_barrier()

        def body_c(j, carry):
            off = pl.multiple_of(blk_b + j * CH, 8)
            pltpu.sync_copy(dst1d.at[pl.ds(off, CH)], idst_b)
            pltpu.sync_copy(ones_v, acc.at[idst_b], add=True)
            return carry

        lax.fori_loop(0, NCHUNK_B, body_c, 0)
        plsc.subcore_barrier()

        @pl.when(c == 0)
        def _wb_c0():
            writeback(cnta)

        @pl.when(c == 1)
        def _wb_c1():
            writeback(cntb)

    return _seg_body


def _make_segsum(with_cnt):
    mesh = plsc.VectorSubcoreMesh(core_axis_name="c", subcore_axis_name="s")
    n_out = 6 if with_cnt else 4
    return pl.kernel(
        _make_seg_body(with_cnt),
        out_type=[jax.ShapeDtypeStruct((NP, H), F32)] * n_out,
        mesh=mesh,
        scratch_types=[
            pltpu.VMEM((2 * GRP, CH), jnp.int32),
            pltpu.VMEM((CH,), jnp.int32),
            pltpu.VMEM((CH,), jnp.int32),
            pltpu.VMEM((CH, H), F32),
            pltpu.VMEM_SHARED((NP, H), F32),
        ],
    )


def _full_spec(shape):
    nd = len(shape)
    return pl.BlockSpec(shape, lambda i, _nd=nd: (0,) * _nd)


def kernel(enrollment, start_date, study_type, time, edge_index, num_W, num_b,
           ln_num_g, ln_num_b, emb, lstm_Wih, lstm_Whh, lstm_bih, lstm_bhh,
           fa_Wi, fa_bi, fa_Wo, fa_bo, conv_W, conv_b, ta_Wi, ta_bi, ta_Wo,
           ta_bo, te_W, te_b, sage1_Wl, sage1_bl, sage1_Wr, sage2_Wl,
           sage2_bl, sage2_Wr, ln1_g, ln1_b, ln2_g, ln2_b, out_W, out_b):
    R = 400
    G = N // R

    src1d = jnp.concatenate(
        [edge_index[0].astype(jnp.int32),
         jnp.arange(EPAD - E, dtype=jnp.int32) % N])
    dst1d = jnp.concatenate(
        [edge_index[1].astype(jnp.int32),
         N + jnp.arange(EPAD - E, dtype=jnp.int32) % (NP - N)])
    enr = enrollment.reshape(N, 1)
    sd = start_date.reshape(N, 1)
    st = study_type.reshape(N, 1).astype(jnp.int32)

    numwt = num_W.reshape(1, H)
    numb = num_b.reshape(1, H)
    lng = ln_num_g.reshape(1, H)
    lnb = ln_num_b.reshape(1, H)
    emb_pad = jnp.pad(emb, ((0, VOCABP - VOCAB), (0, 0)))
    wih = lstm_Wih.reshape(1, 4 * H)
    lbias = (lstm_bih + lstm_bhh).reshape(1, 4 * H)
    fawi = fa_Wi.T
    fabi = fa_bi.reshape(1, 3 * H)
    fawo = fa_Wo.T
    fabo = fa_bo.reshape(1, H)
    convw = jnp.transpose(conv_W, (2, 1, 0))
    convb = conv_b.reshape(1, H)
    tawi = ta_Wi.T
    tabi = ta_bi.reshape(1, 3 * H)
    tawo = ta_Wo.T
    tabo = ta_bo.reshape(1, H)
    tew = te_W.reshape(1, H)
    teb = te_b.reshape(1, H)
    sel = (jnp.arange(H)[:, None] // (H // NH)
           == jnp.arange(NH)[None, :]).astype(F32)
    selt = sel.T

    row_spec = lambda w: pl.BlockSpec((R, w), lambda i: (i, 0))
    weight_args = [numwt, numb, lng, lnb, emb_pad, wih, lbias, fawi, fabi,
                   fawo, fabo, convw, convb, tawi, tabi, tawo, tabo, tew,
                   teb, sel, selt]
    x0, x1, x2 = pl.pallas_call(
        _encode_body,
        grid=(G,),
        in_specs=[row_spec(1), row_spec(1), row_spec(1), row_spec(3)]
        + [_full_spec(w.shape) for w in weight_args],
        out_specs=[row_spec(H)] * 3,
        out_shape=[jax.ShapeDtypeStruct((NP, H), F32)] * 3,
    )(enr, sd, st, time, *weight_args)

    zrows = jnp.zeros((ROWS_PER_SUB, H), F32)
    ones_rows = jnp.ones((CH, H), F32)

    s0, s1, s2a, s2b, cnta, cntb = _make_segsum(True)(
        x0, x1, x2, src1d, dst1d, zrows, ones_rows)

    sage1_w = [sage1_Wl.T, sage1_bl.reshape(1, H), sage1_Wr.T,
               ln1_g.reshape(1, H), ln1_b.reshape(1, H)]
    h0, h1, h2 = pl.pallas_call(
        _sage_body,
        grid=(G,),
        in_specs=[row_spec(H)] * 9
        + [_full_spec(w.shape) for w in sage1_w],
        out_specs=[row_spec(H)] * 3,
        out_shape=[jax.ShapeDtypeStruct((NP, H), F32)] * 3,
    )(s0, s1, s2a, s2b, cnta, cntb, x0, x1, x2, *sage1_w)

    t0, t1, t2a, t2b = _make_segsum(False)(
        h0, h1, h2, src1d, dst1d, zrows, ones_rows)

    sage2_w = [sage2_Wl.T, sage2_bl.reshape(1, H), sage2_Wr.T,
               ln2_g.reshape(1, H), ln2_b.reshape(1, H),
               out_W.T, out_b.reshape(1, H)]
    out = pl.pallas_call(
        _sage_out_body,
        grid=(G,),
        in_specs=[row_spec(H)] * 9
        + [_full_spec(w.shape) for w in sage2_w],
        out_specs=row_spec(3 * H),
        out_shape=jax.ShapeDtypeStruct((N, 3 * H), F32),
    )(t0, t1, t2a, t2b, cnta, cntb, h0, h1, h2, *sage2_w)

    return out.reshape(N, 3, H)


# speedup vs baseline: 1.8308x; 1.0450x over previous
import functools

import jax
import jax.numpy as jnp
import numpy as np
from jax import lax
from jax.experimental import pallas as pl
from jax.experimental.pallas import tpu as pltpu
from jax.experimental.pallas import tpu_sc as plsc

N = 10000
E = 160000
H = 128
NH = 4
VOCAB = 1000
VOCABP = 1024
NP = 10240
NSUB = 16
ROWS_PER_SUB = NP // NSUB
CH = 320
ECHUNKS = 512
EPAD = ECHUNKS * CH
NCHUNK_A = ECHUNKS // NSUB
NCHUNK_B = ECHUNKS // (2 * NSUB)
GRP = 8
NGRP_A = NCHUNK_A // GRP
NGRP_B = NCHUNK_B // GRP

F32 = jnp.float32


def _ln(x, g, b):
    mu = jnp.mean(x, axis=-1, keepdims=True)
    var = jnp.mean((x - mu) ** 2, axis=-1, keepdims=True)
    return (x - mu) / jnp.sqrt(var + 1e-5) * g + b


def _mha(q_in, k_in, v_in, wi_t, bi, wo_t, bo, sel, selt):
    q = [jnp.dot(t, wi_t[:, :H], preferred_element_type=F32) + bi[:, :H]
         for t in q_in]
    k = [jnp.dot(t, wi_t[:, H:2 * H], preferred_element_type=F32) + bi[:, H:2 * H]
         for t in k_in]
    v = [jnp.dot(t, wi_t[:, 2 * H:], preferred_element_type=F32) + bi[:, 2 * H:]
         for t in v_in]
    scale = 1.0 / np.sqrt(H / NH)
    s = [[jnp.dot(q[i] * k[j], sel, preferred_element_type=F32) * scale
          for j in range(3)] for i in range(3)]
    o = []
    for i in range(3):
        m = jnp.maximum(jnp.maximum(s[i][0], s[i][1]), s[i][2])
        e = [jnp.exp(s[i][j] - m) for j in range(3)]
        d = e[0] + e[1] + e[2]
        acc = None
        for j in range(3):
            w = jnp.dot(e[j] / d, selt, preferred_element_type=F32)
            acc = w * v[j] if acc is None else acc + w * v[j]
        o.append(acc)
    return [jnp.dot(t, wo_t, preferred_element_type=F32) + bo for t in o]


def _encode_body(enr_ref, sd_ref, st_ref, time_ref, numwt_ref, numb_ref,
                 lng_ref, lnb_ref, emb_ref, wih_ref, lb_ref, fawi_ref,
                 fabi_ref, fawo_ref, fabo_ref, convw_ref, convb_ref,
                 tawi_ref, tabi_ref, tawo_ref, tabo_ref, tew_ref, teb_ref,
                 sel_ref, selt_ref, x0_ref, x1_ref, x2_ref):
    R = enr_ref.shape[0]
    sel = sel_ref[...]
    selt = selt_ref[...]
    e_num = _ln(enr_ref[...] * numwt_ref[...] + numb_ref[...],
                lng_ref[...], lnb_ref[...])
    e_num = jnp.maximum(e_num, 0.0)
    g = sd_ref[...] * wih_ref[...] + lb_ref[...]
    gi, gg, go = g[:, :H], g[:, 2 * H:3 * H], g[:, 3 * H:]
    cst = jax.nn.sigmoid(gi) * jnp.tanh(gg)
    e_temp = jax.nn.sigmoid(go) * jnp.tanh(cst)
    iota = lax.broadcasted_iota(jnp.int32, (R, VOCABP), 1)
    oh = (iota == st_ref[...]).astype(F32)
    e_cat = jnp.dot(oh, emb_ref[...], preferred_element_type=F32)
    xs = [e_num, e_temp, e_cat]
    xs = _mha(xs, xs, xs, fawi_ref[...], fabi_ref[...], fawo_ref[...],
              fabo_ref[...], sel, selt)
    W = convw_ref[...]
    y = [[jnp.dot(xs[i], W[t], preferred_element_type=F32) for t in range(3)]
         for i in range(3)]
    cb = convb_ref[...]
    xc = [y[0][1] + y[1][2] + cb,
          y[0][0] + y[1][1] + y[2][2] + cb,
          y[1][0] + y[2][1] + cb]
    tm = time_ref[...]
    tew = tew_ref[...]
    teb = teb_ref[...]
    a = [xc[i] + tm[:, i:i + 1] * tew + teb for i in range(3)]
    out = _mha(a, a, xc, tawi_ref[...], tabi_ref[...], tawo_ref[...],
               tabo_ref[...], sel, selt)
    x0_ref[...] = out[0]
    x1_ref[...] = out[1]
    x2_ref[...] = out[2]


def _sage_body(s0_ref, s1_ref, s2a_ref, s2b_ref, ca_ref, cb_ref, x0_ref,
               x1_ref, x2_ref, wlt_ref, bl_ref, wrt_ref, g_ref, b_ref,
               h0_ref, h1_ref, h2_ref):
    cnt = jnp.maximum(ca_ref[...][:, :1] + cb_ref[...][:, :1], 1.0)
    ss = [s0_ref[...], s1_ref[...], s2a_ref[...] + s2b_ref[...]]
    xs = [x0_ref[...], x1_ref[...], x2_ref[...]]
    outs = [h0_ref, h1_ref, h2_ref]
    for i in range(3):
        y = (jnp.dot(ss[i] / cnt, wlt_ref[...], preferred_element_type=F32)
             + bl_ref[...]
             + jnp.dot(xs[i], wrt_ref[...], preferred_element_type=F32))
        outs[i][...] = jnp.maximum(_ln(y, g_ref[...], b_ref[...]), 0.0) + xs[i]


def _sage_out_body(s0_ref, s1_ref, s2a_ref, s2b_ref, ca_ref, cb_ref, x0_ref,
                   x1_ref, x2_ref, wlt_ref, bl_ref, wrt_ref, g_ref, b_ref,
                   owt_ref, ob_ref, out_ref):
    cnt = jnp.maximum(ca_ref[...][:, :1] + cb_ref[...][:, :1], 1.0)
    ss = [s0_ref[...], s1_ref[...], s2a_ref[...] + s2b_ref[...]]
    xs = [x0_ref[...], x1_ref[...], x2_ref[...]]
    cols = []
    for i in range(3):
        y = (jnp.dot(ss[i] / cnt, wlt_ref[...], preferred_element_type=F32)
             + bl_ref[...]
             + jnp.dot(xs[i], wrt_ref[...], preferred_element_type=F32))
        y = jnp.maximum(_ln(y, g_ref[...], b_ref[...]), 0.0) + xs[i]
        cols.append(jnp.dot(y, owt_ref[...], preferred_element_type=F32)
                    + ob_ref[...])
    out_ref[...] = jnp.concatenate(cols, axis=1)


def _make_seg_body(with_cnt):
    def _seg_body(x0, x1, x2, src1d, dst1d, zrows, ones_hbm, *rest):
        if with_cnt:
            (s0, s1, s2a, s2b, cnta, cntb, isrc_b, idst_b, rows0,
             acc) = rest
        else:
            (s0, s1, s2a, s2b, isrc_b, idst_b, rows0, acc) = rest
        c = lax.axis_index("c")
        sid = lax.axis_index("s")
        row0 = pl.multiple_of(sid * ROWS_PER_SUB, 8)

        def zero_acc():
            pltpu.sync_copy(zrows, acc.at[pl.ds(row0, ROWS_PER_SUB)])

        def writeback(dst_hbm):
            pltpu.sync_copy(acc.at[pl.ds(row0, ROWS_PER_SUB)],
                            dst_hbm.at[pl.ds(row0, ROWS_PER_SUB)])

        def run_phase(xtab, nchunks, blk0):
            def body(j, carry):
                off = pl.multiple_of(blk0 + j * CH, 8)
                pltpu.sync_copy(src1d.at[pl.ds(off, CH)], isrc_b)
                pltpu.sync_copy(dst1d.at[pl.ds(off, CH)], idst_b)
                pltpu.sync_copy(xtab.at[isrc_b], rows0)
                pltpu.sync_copy(rows0, acc.at[idst_b], add=True)
                return carry

            lax.fori_loop(0, nchunks, body, 0)

        zero_acc()
        plsc.subcore_barrier()

        blk_a = sid * NCHUNK_A * CH

        @pl.when(c == 0)
        def _pa0():
            run_phase(x0, NCHUNK_A, blk_a)

        @pl.when(c == 1)
        def _pa1():
            run_phase(x1, NCHUNK_A, blk_a)

        plsc.subcore_barrier()

        @pl.when(c == 0)
        def _wb_a0():
            writeback(s0)

        @pl.when(c == 1)
        def _wb_a1():
            writeback(s1)

        zero_acc()
        plsc.subcore_barrier()

        blk_b = (c * (ECHUNKS // 2) + sid * NCHUNK_B) * CH
        run_phase(x2, NCHUNK_B, blk_b)
        plsc.subcore_barrier()

        @pl.when(c == 0)
        def _wb_b0():
            writeback(s2a)

        @pl.when(c == 1)
        def _wb_b1():
            writeback(s2b)

        if not with_cnt:
            return

        zero_acc()
        ones_v = rows0
        pltpu.sync_copy(ones_hbm, ones_v)
        plsc.subcore_barrier()

        def body_c(j, carry):
            off = pl.multiple_of(blk_b + j * CH, 8)
            pltpu.sync_copy(dst1d.at[pl.ds(off, CH)], idst_b)
            pltpu.sync_copy(ones_v, acc.at[idst_b], add=True)
            return carry

        lax.fori_loop(0, NCHUNK_B, body_c, 0)
        plsc.subcore_barrier()

        @pl.when(c == 0)
        def _wb_c0():
            writeback(cnta)

        @pl.when(c == 1)
        def _wb_c1():
            writeback(cntb)

    return _seg_body


def _make_segsum(with_cnt):
    mesh = plsc.VectorSubcoreMesh(core_axis_name="c", subcore_axis_name="s")
    n_out = 6 if with_cnt else 4
    return pl.kernel(
        _make_seg_body(with_cnt),
        out_type=[jax.ShapeDtypeStruct((NP, H), F32)] * n_out,
        mesh=mesh,
        scratch_types=[
            pltpu.VMEM((CH,), jnp.int32),
            pltpu.VMEM((CH,), jnp.int32),
            pltpu.VMEM((CH, H), F32),
            pltpu.VMEM_SHARED((NP, H), F32),
        ],
    )


def _full_spec(shape):
    nd = len(shape)
    return pl.BlockSpec(shape, lambda i, _nd=nd: (0,) * _nd)


def kernel(enrollment, start_date, study_type, time, edge_index, num_W, num_b,
           ln_num_g, ln_num_b, emb, lstm_Wih, lstm_Whh, lstm_bih, lstm_bhh,
           fa_Wi, fa_bi, fa_Wo, fa_bo, conv_W, conv_b, ta_Wi, ta_bi, ta_Wo,
           ta_bo, te_W, te_b, sage1_Wl, sage1_bl, sage1_Wr, sage2_Wl,
           sage2_bl, sage2_Wr, ln1_g, ln1_b, ln2_g, ln2_b, out_W, out_b):
    R = 400
    G = N // R

    src1d = jnp.concatenate(
        [edge_index[0].astype(jnp.int32),
         jnp.arange(EPAD - E, dtype=jnp.int32) % N])
    dst1d = jnp.concatenate(
        [edge_index[1].astype(jnp.int32),
         N + jnp.arange(EPAD - E, dtype=jnp.int32) % (NP - N)])
    enr = enrollment.reshape(N, 1)
    sd = start_date.reshape(N, 1)
    st = study_type.reshape(N, 1).astype(jnp.int32)

    numwt = num_W.reshape(1, H)
    numb = num_b.reshape(1, H)
    lng = ln_num_g.reshape(1, H)
    lnb = ln_num_b.reshape(1, H)
    emb_pad = jnp.pad(emb, ((0, VOCABP - VOCAB), (0, 0)))
    wih = lstm_Wih.reshape(1, 4 * H)
    lbias = (lstm_bih + lstm_bhh).reshape(1, 4 * H)
    fawi = fa_Wi.T
    fabi = fa_bi.reshape(1, 3 * H)
    fawo = fa_Wo.T
    fabo = fa_bo.reshape(1, H)
    convw = jnp.transpose(conv_W, (2, 1, 0))
    convb = conv_b.reshape(1, H)
    tawi = ta_Wi.T
    tabi = ta_bi.reshape(1, 3 * H)
    tawo = ta_Wo.T
    tabo = ta_bo.reshape(1, H)
    tew = te_W.reshape(1, H)
    teb = te_b.reshape(1, H)
    sel = (jnp.arange(H)[:, None] // (H // NH)
           == jnp.arange(NH)[None, :]).astype(F32)
    selt = sel.T

    row_spec = lambda w: pl.BlockSpec((R, w), lambda i: (i, 0))
    weight_args = [numwt, numb, lng, lnb, emb_pad, wih, lbias, fawi, fabi,
                   fawo, fabo, convw, convb, tawi, tabi, tawo, tabo, tew,
                   teb, sel, selt]
    x0, x1, x2 = pl.pallas_call(
        _encode_body,
        grid=(G,),
        in_specs=[row_spec(1), row_spec(1), row_spec(1), row_spec(3)]
        + [_full_spec(w.shape) for w in weight_args],
        out_specs=[row_spec(H)] * 3,
        out_shape=[jax.ShapeDtypeStruct((NP, H), F32)] * 3,
    )(enr, sd, st, time, *weight_args)

    zrows = jnp.zeros((ROWS_PER_SUB, H), F32)
    ones_rows = jnp.ones((CH, H), F32)

    s0, s1, s2a, s2b, cnta, cntb = _make_segsum(True)(
        x0, x1, x2, src1d, dst1d, zrows, ones_rows)

    sage1_w = [sage1_Wl.T, sage1_bl.reshape(1, H), sage1_Wr.T,
               ln1_g.reshape(1, H), ln1_b.reshape(1, H)]
    h0, h1, h2 = pl.pallas_call(
        _sage_body,
        grid=(G,),
        in_specs=[row_spec(H)] * 9
        + [_full_spec(w.shape) for w in sage1_w],
        out_specs=[row_spec(H)] * 3,
        out_shape=[jax.ShapeDtypeStruct((NP, H), F32)] * 3,
    )(s0, s1, s2a, s2b, cnta, cntb, x0, x1, x2, *sage1_w)

    t0, t1, t2a, t2b = _make_segsum(False)(
        h0, h1, h2, src1d, dst1d, zrows, ones_rows)

    sage2_w = [sage2_Wl.T, sage2_bl.reshape(1, H), sage2_Wr.T,
               ln2_g.reshape(1, H), ln2_b.reshape(1, H),
               out_W.T, out_b.reshape(1, H)]
    out = pl.pallas_call(
        _sage_out_body,
        grid=(G,),
        in_specs=[row_spec(H)] * 9
        + [_full_spec(w.shape) for w in sage2_w],
        out_specs=row_spec(3 * H),
        out_shape=jax.ShapeDtypeStruct((N, 3 * H), F32),
    )(t0, t1, t2a, t2b, cnta, cntb, h0, h1, h2, *sage2_w)

    return out.reshape(N, 3, H)
